# Initial kernel scaffold; baseline (speedup 1.0000x reference)
import jax
import jax.numpy as jnp
from jax.experimental import pallas as pl


def kernel(x, hyperedge_index, W1, b1, W2, b2):
    raise NotImplementedError("write your pallas kernel here")


# speedup vs baseline: 21.8070x; 21.8070x over previous
import jax
import jax.numpy as jnp
from jax import lax
from jax.experimental import pallas as pl
from jax.experimental.pallas import tpu as pltpu
from jax.experimental.pallas import tpu_sc as plsc

N_NODES = 10000
NNZ = 320000
R = 10240
ROWS_PER_TILE = 640
CHUNK = 128
N_CHUNKS = NNZ // CHUNK
NUM_TILES = 16
BASE_CHUNKS = N_CHUNKS // NUM_TILES
EXTRA = N_CHUNKS - BASE_CHUNKS * NUM_TILES
SUB = 4
SROWS = ROWS_PER_TILE // SUB

_SC_PARAMS = pltpu.CompilerParams(use_tc_tiling_on_sc=False)


def _matmul_body(x_ref, w_ref, b_ref, o_ref):
    o_ref[...] = (
        jnp.dot(x_ref[...], w_ref[...], preferred_element_type=jnp.float32)
        + b_ref[0]
    )


def _matmul(x, w, b, blk):
    m, k = x.shape
    n = w.shape[1]
    return pl.pallas_call(
        _matmul_body,
        grid=(m // blk,),
        in_specs=[
            pl.BlockSpec((blk, k), lambda i: (i, 0)),
            pl.BlockSpec((k, n), lambda i: (0, 0)),
            pl.BlockSpec((1, n), lambda i: (0, 0)),
        ],
        out_specs=pl.BlockSpec((blk, n), lambda i: (i, 0)),
        out_shape=jax.ShapeDtypeStruct((m, n), jnp.float32),
    )(x, w, b)


def _make_agg(Fc, first_layer):
    mesh = plsc.VectorSubcoreMesh(core_axis_name="c", subcore_axis_name="s")

    if first_layer:
        out_type = (
            jax.ShapeDtypeStruct((R, 128), jnp.float32),
            jax.ShapeDtypeStruct((R,), jnp.float32),
            jax.ShapeDtypeStruct((R,), jnp.float32),
        )
    else:
        out_type = jax.ShapeDtypeStruct((R, 128), jnp.float32)

    scratch = dict(
        in_table=pltpu.VMEM_SHARED((R, Fc), jnp.float32),
        edge_acc=pltpu.VMEM_SHARED((R, Fc), jnp.float32),
        nbuf=pltpu.VMEM((CHUNK,), jnp.int32),
        ebuf=pltpu.VMEM((CHUNK,), jnp.int32),
        rows=pltpu.VMEM((CHUNK, Fc), jnp.float32),
        accbuf=pltpu.VMEM((SROWS, Fc), jnp.float32),
        degbuf=pltpu.VMEM((ROWS_PER_TILE,), jnp.float32),
        onesv=pltpu.VMEM((CHUNK,), jnp.float32),
    )
    if first_layer:
        scratch["bdeg"] = pltpu.VMEM_SHARED((R,), jnp.float32)
        scratch["ddeg"] = pltpu.VMEM_SHARED((R,), jnp.float32)

    def body(*refs, in_table, edge_acc, nbuf, ebuf, rows, accbuf,
             degbuf, onesv, bdeg=None, ddeg=None):
        if first_layer:
            y_h, nidx_h, eidx_h, h_out, binv_out, dinv_out = refs
        else:
            y_h, nidx_h, eidx_h, binv_h, dinv_h, out_h = refs

        c = lax.axis_index("c")
        t = lax.axis_index("s")
        r0 = t * ROWS_PER_TILE
        cnt = BASE_CHUNKS + jnp.where(t < EXTRA, 1, 0)
        c0 = t * BASE_CHUNKS + jnp.minimum(t, EXTRA)
        col0 = c * Fc
        rslice = pl.ds(r0, ROWS_PER_TILE)

        def zrow_f(i, carry):
            for jj in range(Fc // 16):
                accbuf[i, pl.ds(jj * 16, 16)] = jnp.zeros((16,), jnp.float32)
            return carry

        lax.fori_loop(0, SROWS, zrow_f, 0)

        def zdeg_f(i, carry):
            degbuf[pl.ds(i * 16, 16)] = jnp.zeros((16,), jnp.float32)
            return carry

        lax.fori_loop(0, ROWS_PER_TILE // 16, zdeg_f, 0)

        for jj in range(CHUNK // 16):
            onesv[pl.ds(jj * 16, 16)] = jnp.ones((16,), jnp.float32)

        for sb in range(SUB):
            pltpu.sync_copy(accbuf, edge_acc.at[pl.ds(r0 + sb * SROWS, SROWS)])
        if first_layer:
            pltpu.sync_copy(degbuf, bdeg.at[rslice])
            pltpu.sync_copy(degbuf, ddeg.at[rslice])
        pltpu.sync_copy(y_h.at[rslice, pl.ds(col0, Fc)], in_table.at[rslice])
        plsc.subcore_barrier()

        def hop1(j, carry):
            off = (c0 + j) * CHUNK
            pltpu.sync_copy(nidx_h.at[pl.ds(off, CHUNK)], nbuf)
            pltpu.sync_copy(eidx_h.at[pl.ds(off, CHUNK)], ebuf)
            pltpu.sync_copy(in_table.at[nbuf], rows)
            pltpu.sync_copy(rows, edge_acc.at[ebuf], add=True)
            if first_layer:
                pltpu.sync_copy(onesv, bdeg.at[ebuf], add=True)
                pltpu.sync_copy(onesv, ddeg.at[nbuf], add=True)
            return carry

        lax.fori_loop(0, cnt, hop1, 0)
        plsc.subcore_barrier()

        for sb in range(SUB):
            pltpu.sync_copy(accbuf, in_table.at[pl.ds(r0 + sb * SROWS, SROWS)])
        if first_layer:
            pltpu.sync_copy(bdeg.at[rslice], degbuf)

            def inv_g(g, carry):
                sl = pl.ds(g * 16, 16)
                d = degbuf[sl]
                degbuf[sl] = jnp.where(d > 0.0, 1.0 / d, 0.0)
                return carry

            lax.fori_loop(0, ROWS_PER_TILE // 16, inv_g, 0)
        else:
            pltpu.sync_copy(binv_h.at[rslice], degbuf)

        for sb in range(SUB):
            sbslice = pl.ds(r0 + sb * SROWS, SROWS)
            pltpu.sync_copy(edge_acc.at[sbslice], accbuf)

            def scale_r(g, carry, sb=sb):
                deg16 = degbuf[pl.ds(sb * SROWS + g * 16, 16)]
                for r16 in range(16):
                    bs = deg16[r16]
                    row = g * 16 + r16
                    for jj in range(Fc // 16):
                        sl = pl.ds(jj * 16, 16)
                        accbuf[row, sl] = accbuf[row, sl] * bs
                return carry

            lax.fori_loop(0, SROWS // 16, scale_r, 0)
            pltpu.sync_copy(accbuf, edge_acc.at[sbslice])
        if first_layer:

            @pl.when(c == 0)
            def _():
                pltpu.sync_copy(degbuf, binv_out.at[rslice])

        plsc.subcore_barrier()

        def hop2(j, carry):
            off = (c0 + j) * CHUNK
            pltpu.sync_copy(nidx_h.at[pl.ds(off, CHUNK)], nbuf)
            pltpu.sync_copy(eidx_h.at[pl.ds(off, CHUNK)], ebuf)
            pltpu.sync_copy(edge_acc.at[ebuf], rows)
            pltpu.sync_copy(rows, in_table.at[nbuf], add=True)
            return carry

        lax.fori_loop(0, cnt, hop2, 0)
        plsc.subcore_barrier()

        if first_layer:
            pltpu.sync_copy(ddeg.at[rslice], degbuf)

            def inv_g2(g, carry):
                sl = pl.ds(g * 16, 16)
                d = degbuf[sl]
                degbuf[sl] = jnp.where(d > 0.0, 1.0 / d, 0.0)
                return carry

            lax.fori_loop(0, ROWS_PER_TILE // 16, inv_g2, 0)
        else:
            pltpu.sync_copy(dinv_h.at[rslice], degbuf)

        dst = h_out if first_layer else out_h
        for sb in range(SUB):
            sbslice = pl.ds(r0 + sb * SROWS, SROWS)
            pltpu.sync_copy(in_table.at[sbslice], accbuf)

            def scale_out(g, carry, sb=sb):
                deg16 = degbuf[pl.ds(sb * SROWS + g * 16, 16)]
                for r16 in range(16):
                    ds_ = deg16[r16]
                    row = g * 16 + r16
                    for jj in range(Fc // 16):
                        sl = pl.ds(jj * 16, 16)
                        v = accbuf[row, sl] * ds_
                        if first_layer:
                            v = jnp.where(
                                v > 0.0, v, jnp.exp(jnp.minimum(v, 0.0)) - 1.0
                            )
                        accbuf[row, sl] = v
                return carry

            lax.fori_loop(0, SROWS // 16, scale_out, 0)
            pltpu.sync_copy(accbuf, dst.at[sbslice, pl.ds(col0, Fc)])
        if first_layer:

            @pl.when(c == 0)
            def _():
                pltpu.sync_copy(degbuf, dinv_out.at[rslice])

    return pl.kernel(
        body,
        out_type=out_type,
        mesh=mesh,
        scratch_types=scratch,
        compiler_params=_SC_PARAMS,
    )


@jax.jit
def kernel(x, hyperedge_index, W1, b1, W2, b2):
    node_idx = hyperedge_index[0]
    edge_idx = hyperedge_index[1]

    num_heads, in_dim, head_dim = W1.shape
    hidden = num_heads * head_dim
    num_classes = W2.shape[1]

    w1cat = W1.transpose(1, 0, 2).reshape(in_dim, hidden)
    b1cat = b1.reshape(1, hidden)
    w2p = jnp.pad(W2, ((0, 0), (0, 128 - num_classes)))
    b2p = jnp.pad(b2, (0, 128 - num_classes)).reshape(1, 128)

    xp = jnp.pad(x, ((0, R - x.shape[0]), (0, 0)))
    y1 = _matmul(xp, w1cat, b1cat, 640)
    agg1 = _make_agg(hidden // 2, True)
    h_pad, binv, dinv = agg1(y1, node_idx, edge_idx)

    y2 = _matmul(h_pad, w2p, b2p, 640)
    agg2 = _make_agg(num_classes // 2, False)
    out_pad = agg2(y2, node_idx, edge_idx, binv, dinv)

    return out_pad[:N_NODES, :num_classes]


# speedup vs baseline: 43.4271x; 1.9914x over previous
import jax
import jax.numpy as jnp
from jax import lax
from jax.experimental import pallas as pl
from jax.experimental.pallas import tpu as pltpu
from jax.experimental.pallas import tpu_sc as plsc

N_NODES = 10000
NNZ = 320000
R = 10240
ROWS_PER_TILE = 640
CHUNK = 128
N_CHUNKS = NNZ // CHUNK
NUM_TILES = 16
BASE_CHUNKS = N_CHUNKS // NUM_TILES
EXTRA = N_CHUNKS - BASE_CHUNKS * NUM_TILES
SUB = 4
SROWS = ROWS_PER_TILE // SUB

_SC_PARAMS = pltpu.CompilerParams(use_tc_tiling_on_sc=False)


def _matmul_body(x_ref, w_ref, b_ref, o_ref):
    o_ref[...] = (
        jnp.dot(x_ref[...], w_ref[...], preferred_element_type=jnp.float32)
        + b_ref[0]
    )


def _matmul(x, w, b, blk):
    m, k = x.shape
    n = w.shape[1]
    return pl.pallas_call(
        _matmul_body,
        grid=(m // blk,),
        in_specs=[
            pl.BlockSpec((blk, k), lambda i: (i, 0)),
            pl.BlockSpec((k, n), lambda i: (0, 0)),
            pl.BlockSpec((1, n), lambda i: (0, 0)),
        ],
        out_specs=pl.BlockSpec((blk, n), lambda i: (i, 0)),
        out_shape=jax.ShapeDtypeStruct((m, n), jnp.float32),
    )(x, w, b)


def _make_agg(Fc, first_layer):
    mesh = plsc.VectorSubcoreMesh(core_axis_name="c", subcore_axis_name="s")

    if first_layer:
        out_type = (
            jax.ShapeDtypeStruct((R, 128), jnp.float32),
            jax.ShapeDtypeStruct((R,), jnp.float32),
            jax.ShapeDtypeStruct((R,), jnp.float32),
        )
    else:
        out_type = jax.ShapeDtypeStruct((R, 128), jnp.float32)

    scratch = dict(
        in_table=pltpu.VMEM_SHARED((R, Fc), jnp.float32),
        edge_acc=pltpu.VMEM_SHARED((R, Fc), jnp.float32),
        nbufs=pltpu.VMEM((3, CHUNK), jnp.int32),
        ebufs=pltpu.VMEM((3, CHUNK), jnp.int32),
        rowss=pltpu.VMEM((3, CHUNK, Fc), jnp.float32),
        accbuf=pltpu.VMEM((SROWS, Fc), jnp.float32),
        degbuf=pltpu.VMEM((ROWS_PER_TILE,), jnp.float32),
        onesv=pltpu.VMEM((CHUNK,), jnp.float32),
        semi=pltpu.SemaphoreType.DMA((3,)),
        semg=pltpu.SemaphoreType.DMA((3,)),
        sems=pltpu.SemaphoreType.DMA((3,)),
        semo=pltpu.SemaphoreType.DMA((3,)),
    )
    if first_layer:
        scratch["bdeg"] = pltpu.VMEM_SHARED((R,), jnp.float32)
        scratch["ddeg"] = pltpu.VMEM_SHARED((R,), jnp.float32)

    def body(*refs, in_table, edge_acc, nbufs, ebufs, rowss, accbuf,
             degbuf, onesv, semi, semg, sems, semo, bdeg=None, ddeg=None):
        if first_layer:
            y_h, nidx_h, eidx_h, h_out, binv_out, dinv_out = refs
        else:
            y_h, nidx_h, eidx_h, binv_h, dinv_h, out_h = refs

        c = lax.axis_index("c")
        t = lax.axis_index("s")
        r0 = t * ROWS_PER_TILE
        cnt = BASE_CHUNKS + jnp.where(t < EXTRA, 1, 0)
        c0 = t * BASE_CHUNKS + jnp.minimum(t, EXTRA)
        col0 = c * Fc
        rslice = pl.ds(r0, ROWS_PER_TILE)

        def zrow_f(i, carry):
            for jj in range(Fc // 16):
                accbuf[i, pl.ds(jj * 16, 16)] = jnp.zeros((16,), jnp.float32)
            return carry

        lax.fori_loop(0, SROWS, zrow_f, 0)

        def zdeg_f(i, carry):
            degbuf[pl.ds(i * 16, 16)] = jnp.zeros((16,), jnp.float32)
            return carry

        lax.fori_loop(0, ROWS_PER_TILE // 16, zdeg_f, 0)

        for jj in range(CHUNK // 16):
            onesv[pl.ds(jj * 16, 16)] = jnp.ones((16,), jnp.float32)

        for sb in range(SUB):
            pltpu.sync_copy(accbuf, edge_acc.at[pl.ds(r0 + sb * SROWS, SROWS)])
        if first_layer:
            pltpu.sync_copy(degbuf, bdeg.at[rslice])
            pltpu.sync_copy(degbuf, ddeg.at[rslice])
        pltpu.sync_copy(y_h.at[rslice, pl.ds(col0, Fc)], in_table.at[rslice])
        plsc.subcore_barrier()

        def run_hop(gsrc, gidx_bufs, sdst, sidx_bufs, do_deg):
            def issue_idx(j, k):
                off = (c0 + j) * CHUNK
                pltpu.async_copy(
                    nidx_h.at[pl.ds(off, CHUNK)], nbufs.at[k], semi.at[k]
                )
                pltpu.async_copy(
                    eidx_h.at[pl.ds(off, CHUNK)], ebufs.at[k], semi.at[k]
                )

            def wait_idx(k):
                pltpu.make_async_copy(
                    nidx_h.at[pl.ds(0, CHUNK)], nbufs.at[k], semi.at[k]
                ).wait()
                pltpu.make_async_copy(
                    eidx_h.at[pl.ds(0, CHUNK)], ebufs.at[k], semi.at[k]
                ).wait()

            def issue_scatter(k):
                pltpu.async_copy(
                    rowss.at[k], sdst.at[sidx_bufs.at[k]], sems.at[k],
                    add=True,
                )
                if do_deg:
                    pltpu.async_copy(
                        onesv, bdeg.at[ebufs.at[k]], semo.at[k], add=True
                    )
                    pltpu.async_copy(
                        onesv, ddeg.at[nbufs.at[k]], semo.at[k], add=True
                    )

            def wait_scatter(k):
                pltpu.make_async_copy(
                    rowss.at[k], sdst.at[sidx_bufs.at[k]], sems.at[k]
                ).wait()
                if do_deg:
                    pltpu.make_async_copy(
                        onesv, bdeg.at[ebufs.at[k]], semo.at[k]
                    ).wait()
                    pltpu.make_async_copy(
                        onesv, ddeg.at[nbufs.at[k]], semo.at[k]
                    ).wait()

            issue_idx(0, 0)
            issue_idx(1, 1)

            def step(g, carry):
                for k in range(3):
                    j = 3 * g + k
                    k2 = (k + 2) % 3

                    @pl.when(j < cnt)
                    def _(j=j, k=k, k2=k2):
                        wait_idx(k)
                        pltpu.async_copy(
                            gsrc.at[gidx_bufs.at[k]], rowss.at[k], semg.at[k]
                        ).wait()
                        issue_scatter(k)

                        @pl.when(j >= 1)
                        def _():
                            wait_scatter(k2)

                        @pl.when(j + 2 < cnt)
                        def _():
                            issue_idx(j + 2, k2)

                return carry

            lax.fori_loop(0, (BASE_CHUNKS + 1 + 2) // 3, step, 0)
            for k in range(3):

                @pl.when((cnt - 1) % 3 == k)
                def _(k=k):
                    wait_scatter(k)

        run_hop(in_table, nbufs, edge_acc, ebufs, first_layer)
        plsc.subcore_barrier()

        for sb in range(SUB):
            pltpu.sync_copy(accbuf, in_table.at[pl.ds(r0 + sb * SROWS, SROWS)])
        if first_layer:
            pltpu.sync_copy(bdeg.at[rslice], degbuf)

            def inv_g(g, carry):
                sl = pl.ds(g * 16, 16)
                d = degbuf[sl]
                degbuf[sl] = jnp.where(d > 0.0, 1.0 / d, 0.0)
                return carry

            lax.fori_loop(0, ROWS_PER_TILE // 16, inv_g, 0)
        else:
            pltpu.sync_copy(binv_h.at[rslice], degbuf)

        for sb in range(SUB):
            sbslice = pl.ds(r0 + sb * SROWS, SROWS)
            pltpu.sync_copy(edge_acc.at[sbslice], accbuf)

            def scale_r(g, carry, sb=sb):
                deg16 = degbuf[pl.ds(sb * SROWS + g * 16, 16)]
                for r16 in range(16):
                    bs = deg16[r16]
                    row = g * 16 + r16
                    for jj in range(Fc // 16):
                        sl = pl.ds(jj * 16, 16)
                        accbuf[row, sl] = accbuf[row, sl] * bs
                return carry

            lax.fori_loop(0, SROWS // 16, scale_r, 0)
            pltpu.sync_copy(accbuf, edge_acc.at[sbslice])
        if first_layer:

            @pl.when(c == 0)
            def _():
                pltpu.sync_copy(degbuf, binv_out.at[rslice])

        plsc.subcore_barrier()

        run_hop(edge_acc, ebufs, in_table, nbufs, False)
        plsc.subcore_barrier()

        if first_layer:
            pltpu.sync_copy(ddeg.at[rslice], degbuf)

            def inv_g2(g, carry):
                sl = pl.ds(g * 16, 16)
                d = degbuf[sl]
                degbuf[sl] = jnp.where(d > 0.0, 1.0 / d, 0.0)
                return carry

            lax.fori_loop(0, ROWS_PER_TILE // 16, inv_g2, 0)
        else:
            pltpu.sync_copy(dinv_h.at[rslice], degbuf)

        dst = h_out if first_layer else out_h
        for sb in range(SUB):
            sbslice = pl.ds(r0 + sb * SROWS, SROWS)
            pltpu.sync_copy(in_table.at[sbslice], accbuf)

            def scale_out(g, carry, sb=sb):
                deg16 = degbuf[pl.ds(sb * SROWS + g * 16, 16)]
                for r16 in range(16):
                    ds_ = deg16[r16]
                    row = g * 16 + r16
                    for jj in range(Fc // 16):
                        sl = pl.ds(jj * 16, 16)
                        v = accbuf[row, sl] * ds_
                        if first_layer:
                            v = jnp.where(
                                v > 0.0, v, jnp.exp(jnp.minimum(v, 0.0)) - 1.0
                            )
                        accbuf[row, sl] = v
                return carry

            lax.fori_loop(0, SROWS // 16, scale_out, 0)
            pltpu.sync_copy(accbuf, dst.at[sbslice, pl.ds(col0, Fc)])
        if first_layer:

            @pl.when(c == 0)
            def _():
                pltpu.sync_copy(degbuf, dinv_out.at[rslice])

    return pl.kernel(
        body,
        out_type=out_type,
        mesh=mesh,
        scratch_types=scratch,
        compiler_params=_SC_PARAMS,
    )


@jax.jit
def kernel(x, hyperedge_index, W1, b1, W2, b2):
    node_idx = hyperedge_index[0]
    edge_idx = hyperedge_index[1]

    num_heads, in_dim, head_dim = W1.shape
    hidden = num_heads * head_dim
    num_classes = W2.shape[1]

    w1cat = W1.transpose(1, 0, 2).reshape(in_dim, hidden)
    b1cat = b1.reshape(1, hidden)
    w2p = jnp.pad(W2, ((0, 0), (0, 128 - num_classes)))
    b2p = jnp.pad(b2, (0, 128 - num_classes)).reshape(1, 128)

    xp = jnp.pad(x, ((0, R - x.shape[0]), (0, 0)))
    y1 = _matmul(xp, w1cat, b1cat, 640)
    agg1 = _make_agg(hidden // 2, True)
    h_pad, binv, dinv = agg1(y1, node_idx, edge_idx)

    y2 = _matmul(h_pad, w2p, b2p, 640)
    agg2 = _make_agg(num_classes // 2, False)
    out_pad = agg2(y2, node_idx, edge_idx, binv, dinv)

    return out_pad[:N_NODES, :num_classes]


# speedup vs baseline: 51.5366x; 1.1867x over previous
import jax
import jax.numpy as jnp
from jax import lax
from jax.experimental import pallas as pl
from jax.experimental.pallas import tpu as pltpu
from jax.experimental.pallas import tpu_sc as plsc

N_NODES = 10000
NNZ = 320000
R = 10240
ROWS_PER_TILE = 640
CHUNK = 128
N_CHUNKS = NNZ // CHUNK
NUM_TILES = 16
BASE_CHUNKS = N_CHUNKS // NUM_TILES
EXTRA = N_CHUNKS - BASE_CHUNKS * NUM_TILES
SUB = 4
SROWS = ROWS_PER_TILE // SUB

_SC_PARAMS = pltpu.CompilerParams(use_tc_tiling_on_sc=False)


def _matmul_body(x_ref, w_ref, b_ref, o_ref):
    o_ref[...] = (
        jnp.dot(x_ref[...], w_ref[...], preferred_element_type=jnp.float32)
        + b_ref[0]
    )


def _matmul(x, w, b, blk):
    m, k = x.shape
    n = w.shape[1]
    return pl.pallas_call(
        _matmul_body,
        grid=(m // blk,),
        in_specs=[
            pl.BlockSpec((blk, k), lambda i: (i, 0)),
            pl.BlockSpec((k, n), lambda i: (0, 0)),
            pl.BlockSpec((1, n), lambda i: (0, 0)),
        ],
        out_specs=pl.BlockSpec((blk, n), lambda i: (i, 0)),
        out_shape=jax.ShapeDtypeStruct((m, n), jnp.float32),
    )(x, w, b)


def _make_agg(Fc, first_layer):
    mesh = plsc.VectorSubcoreMesh(core_axis_name="c", subcore_axis_name="s")

    if first_layer:
        out_type = (
            jax.ShapeDtypeStruct((R, 128), jnp.float32),
            jax.ShapeDtypeStruct((R,), jnp.float32),
            jax.ShapeDtypeStruct((R,), jnp.float32),
        )
    else:
        out_type = jax.ShapeDtypeStruct((R, 128), jnp.float32)

    scratch = dict(
        in_table=pltpu.VMEM_SHARED((R, Fc), jnp.float32),
        edge_acc=pltpu.VMEM_SHARED((R, Fc), jnp.float32),
        nbufs=pltpu.VMEM((4, CHUNK), jnp.int32),
        ebufs=pltpu.VMEM((4, CHUNK), jnp.int32),
        rowss=pltpu.VMEM((4, CHUNK, Fc), jnp.float32),
        accbuf=pltpu.VMEM((SROWS, Fc), jnp.float32),
        degbuf=pltpu.VMEM((ROWS_PER_TILE,), jnp.float32),
        onesv=pltpu.VMEM((CHUNK,), jnp.float32),
        semi=pltpu.SemaphoreType.DMA((4,)),
        semg=pltpu.SemaphoreType.DMA((4,)),
        sems=pltpu.SemaphoreType.DMA((4,)),
        semo=pltpu.SemaphoreType.DMA((4,)),
    )
    if first_layer:
        scratch["bdeg"] = pltpu.VMEM_SHARED((R,), jnp.float32)
        scratch["ddeg"] = pltpu.VMEM_SHARED((R,), jnp.float32)

    def body(*refs, in_table, edge_acc, nbufs, ebufs, rowss, accbuf,
             degbuf, onesv, semi, semg, sems, semo, bdeg=None, ddeg=None):
        if first_layer:
            y_h, nidx_h, eidx_h, h_out, binv_out, dinv_out = refs
        else:
            y_h, nidx_h, eidx_h, binv_h, dinv_h, out_h = refs

        c = lax.axis_index("c")
        t = lax.axis_index("s")
        r0 = t * ROWS_PER_TILE
        cnt = BASE_CHUNKS + jnp.where(t < EXTRA, 1, 0)
        c0 = t * BASE_CHUNKS + jnp.minimum(t, EXTRA)
        col0 = c * Fc
        rslice = pl.ds(r0, ROWS_PER_TILE)

        def zrow_f(i, carry):
            for jj in range(Fc // 16):
                accbuf[i, pl.ds(jj * 16, 16)] = jnp.zeros((16,), jnp.float32)
            return carry

        lax.fori_loop(0, SROWS, zrow_f, 0)

        def zdeg_f(i, carry):
            degbuf[pl.ds(i * 16, 16)] = jnp.zeros((16,), jnp.float32)
            return carry

        lax.fori_loop(0, ROWS_PER_TILE // 16, zdeg_f, 0)

        for jj in range(CHUNK // 16):
            onesv[pl.ds(jj * 16, 16)] = jnp.ones((16,), jnp.float32)

        for sb in range(SUB):
            pltpu.sync_copy(accbuf, edge_acc.at[pl.ds(r0 + sb * SROWS, SROWS)])
        if first_layer:
            pltpu.sync_copy(degbuf, bdeg.at[rslice])
            pltpu.sync_copy(degbuf, ddeg.at[rslice])
        pltpu.sync_copy(y_h.at[rslice, pl.ds(col0, Fc)], in_table.at[rslice])
        plsc.subcore_barrier()

        def run_hop(gsrc, gidx_bufs, sdst, sidx_bufs, deg_tbl, deg_bufs):
            def issue_idx(j, k):
                off = (c0 + j) * CHUNK
                pltpu.async_copy(
                    nidx_h.at[pl.ds(off, CHUNK)], nbufs.at[k], semi.at[k]
                )
                pltpu.async_copy(
                    eidx_h.at[pl.ds(off, CHUNK)], ebufs.at[k], semi.at[k]
                )

            def wait_idx(k):
                pltpu.make_async_copy(
                    nidx_h.at[pl.ds(0, CHUNK)], nbufs.at[k], semi.at[k]
                ).wait()
                pltpu.make_async_copy(
                    eidx_h.at[pl.ds(0, CHUNK)], ebufs.at[k], semi.at[k]
                ).wait()

            def issue_scatter(k):
                pltpu.async_copy(
                    rowss.at[k], sdst.at[sidx_bufs.at[k]], sems.at[k],
                    add=True,
                )
                if deg_tbl is not None:
                    pltpu.async_copy(
                        onesv, deg_tbl.at[deg_bufs.at[k]], semo.at[k],
                        add=True,
                    )

            def wait_scatter(k):
                pltpu.make_async_copy(
                    rowss.at[k], sdst.at[sidx_bufs.at[k]], sems.at[k]
                ).wait()
                if deg_tbl is not None:
                    pltpu.make_async_copy(
                        onesv, deg_tbl.at[deg_bufs.at[k]], semo.at[k]
                    ).wait()

            issue_idx(0, 0)
            issue_idx(1, 1)

            def step4(g, carry):
                for k in range(4):
                    j = 4 * g + k
                    k1 = (k + 3) % 4
                    k2 = (k + 2) % 4

                    @pl.when(j < cnt)
                    def _(j=j, k=k):
                        wait_idx(k)
                        pltpu.async_copy(
                            gsrc.at[gidx_bufs.at[k]], rowss.at[k], semg.at[k]
                        )

                    @pl.when(jnp.logical_and(j >= 1, j <= cnt))
                    def _(j=j, k1=k1):
                        pltpu.make_async_copy(
                            gsrc.at[gidx_bufs.at[k1]], rowss.at[k1],
                            semg.at[k1],
                        ).wait()
                        issue_scatter(k1)

                    @pl.when(jnp.logical_and(j >= 2, j <= cnt + 1))
                    def _(j=j, k2=k2):
                        wait_scatter(k2)

                    @pl.when(j + 2 < cnt)
                    def _(j=j, k2=k2):
                        issue_idx(j + 2, k2)

                return carry

            lax.fori_loop(0, (BASE_CHUNKS + 1 + 2 + 3) // 4, step4, 0)

        run_hop(in_table, nbufs, edge_acc, ebufs,
                bdeg if first_layer else None, ebufs)
        plsc.subcore_barrier()

        for sb in range(SUB):
            pltpu.sync_copy(accbuf, in_table.at[pl.ds(r0 + sb * SROWS, SROWS)])
        if first_layer:
            pltpu.sync_copy(bdeg.at[rslice], degbuf)

            def inv_g(g, carry):
                sl = pl.ds(g * 16, 16)
                d = degbuf[sl]
                degbuf[sl] = jnp.where(d > 0.0, 1.0 / d, 0.0)
                return carry

            lax.fori_loop(0, ROWS_PER_TILE // 16, inv_g, 0)
        else:
            pltpu.sync_copy(binv_h.at[rslice], degbuf)

        for sb in range(SUB):
            sbslice = pl.ds(r0 + sb * SROWS, SROWS)
            pltpu.sync_copy(edge_acc.at[sbslice], accbuf)

            def scale_r(g, carry, sb=sb):
                deg16 = degbuf[pl.ds(sb * SROWS + g * 16, 16)]
                for r16 in range(16):
                    bs = deg16[r16]
                    row = g * 16 + r16
                    for jj in range(Fc // 16):
                        sl = pl.ds(jj * 16, 16)
                        accbuf[row, sl] = accbuf[row, sl] * bs
                return carry

            lax.fori_loop(0, SROWS // 16, scale_r, 0)
            pltpu.sync_copy(accbuf, edge_acc.at[sbslice])
        if first_layer:

            @pl.when(c == 0)
            def _():
                pltpu.sync_copy(degbuf, binv_out.at[rslice])

        plsc.subcore_barrier()

        run_hop(edge_acc, ebufs, in_table, nbufs,
                ddeg if first_layer else None, nbufs)
        plsc.subcore_barrier()

        if first_layer:
            pltpu.sync_copy(ddeg.at[rslice], degbuf)

            def inv_g2(g, carry):
                sl = pl.ds(g * 16, 16)
                d = degbuf[sl]
                degbuf[sl] = jnp.where(d > 0.0, 1.0 / d, 0.0)
                return carry

            lax.fori_loop(0, ROWS_PER_TILE // 16, inv_g2, 0)
        else:
            pltpu.sync_copy(dinv_h.at[rslice], degbuf)

        dst = h_out if first_layer else out_h
        for sb in range(SUB):
            sbslice = pl.ds(r0 + sb * SROWS, SROWS)
            pltpu.sync_copy(in_table.at[sbslice], accbuf)

            def scale_out(g, carry, sb=sb):
                deg16 = degbuf[pl.ds(sb * SROWS + g * 16, 16)]
                for r16 in range(16):
                    ds_ = deg16[r16]
                    row = g * 16 + r16
                    for jj in range(Fc // 16):
                        sl = pl.ds(jj * 16, 16)
                        v = accbuf[row, sl] * ds_
                        if first_layer:
                            v = jnp.where(
                                v > 0.0, v, jnp.exp(jnp.minimum(v, 0.0)) - 1.0
                            )
                        accbuf[row, sl] = v
                return carry

            lax.fori_loop(0, SROWS // 16, scale_out, 0)
            pltpu.sync_copy(accbuf, dst.at[sbslice, pl.ds(col0, Fc)])
        if first_layer:

            @pl.when(c == 0)
            def _():
                pltpu.sync_copy(degbuf, dinv_out.at[rslice])

    return pl.kernel(
        body,
        out_type=out_type,
        mesh=mesh,
        scratch_types=scratch,
        compiler_params=_SC_PARAMS,
    )


@jax.jit
def kernel(x, hyperedge_index, W1, b1, W2, b2):
    node_idx = hyperedge_index[0]
    edge_idx = hyperedge_index[1]

    num_heads, in_dim, head_dim = W1.shape
    hidden = num_heads * head_dim
    num_classes = W2.shape[1]

    w1cat = W1.transpose(1, 0, 2).reshape(in_dim, hidden)
    b1cat = b1.reshape(1, hidden)
    w2p = jnp.pad(W2, ((0, 0), (0, 128 - num_classes)))
    b2p = jnp.pad(b2, (0, 128 - num_classes)).reshape(1, 128)

    xp = jnp.pad(x, ((0, R - x.shape[0]), (0, 0)))
    y1 = _matmul(xp, w1cat, b1cat, 640)
    agg1 = _make_agg(hidden // 2, True)
    h_pad, binv, dinv = agg1(y1, node_idx, edge_idx)

    y2 = _matmul(h_pad, w2p, b2p, 640)
    agg2 = _make_agg(num_classes // 2, False)
    out_pad = agg2(y2, node_idx, edge_idx, binv, dinv)

    return out_pad[:N_NODES, :num_classes]


# speedup vs baseline: 51.5876x; 1.0010x over previous
import jax
import jax.numpy as jnp
from jax import lax
from jax.experimental import pallas as pl
from jax.experimental.pallas import tpu as pltpu
from jax.experimental.pallas import tpu_sc as plsc

N_NODES = 10000
NNZ = 320000
R = 10240
ROWS_PER_TILE = 640
NUM_TILES = 16
SUB = 4
SROWS = ROWS_PER_TILE // SUB

_SC_PARAMS = pltpu.CompilerParams(use_tc_tiling_on_sc=False)


def _matmul_body(x_ref, w_ref, b_ref, o_ref):
    o_ref[...] = (
        jnp.dot(x_ref[...], w_ref[...], preferred_element_type=jnp.float32)
        + b_ref[0]
    )


def _matmul(x, w, b, blk):
    m, k = x.shape
    n = w.shape[1]
    return pl.pallas_call(
        _matmul_body,
        grid=(m // blk,),
        in_specs=[
            pl.BlockSpec((blk, k), lambda i: (i, 0)),
            pl.BlockSpec((k, n), lambda i: (0, 0)),
            pl.BlockSpec((1, n), lambda i: (0, 0)),
        ],
        out_specs=pl.BlockSpec((blk, n), lambda i: (i, 0)),
        out_shape=jax.ShapeDtypeStruct((m, n), jnp.float32),
    )(x, w, b)


def _make_agg(Fc, first_layer, chunk):
    mesh = plsc.VectorSubcoreMesh(core_axis_name="c", subcore_axis_name="s")
    n_chunks = NNZ // chunk
    base = n_chunks // NUM_TILES
    extra = n_chunks - base * NUM_TILES

    if first_layer:
        out_type = (
            jax.ShapeDtypeStruct((R, 128), jnp.float32),
            jax.ShapeDtypeStruct((R,), jnp.float32),
            jax.ShapeDtypeStruct((R,), jnp.float32),
        )
    else:
        out_type = jax.ShapeDtypeStruct((R, 128), jnp.float32)

    scratch = dict(
        in_table=pltpu.VMEM_SHARED((R, Fc), jnp.float32),
        edge_acc=pltpu.VMEM_SHARED((R, Fc), jnp.float32),
        ibufs=pltpu.VMEM((4, 2, chunk), jnp.int32),
        rowss=pltpu.VMEM((4, chunk, Fc), jnp.float32),
        accbuf=pltpu.VMEM((SROWS, Fc), jnp.float32),
        degbuf=pltpu.VMEM((ROWS_PER_TILE,), jnp.float32),
        onesv=pltpu.VMEM((chunk,), jnp.float32),
        semi=pltpu.SemaphoreType.DMA((4,)),
        semg=pltpu.SemaphoreType.DMA((4,)),
        sems=pltpu.SemaphoreType.DMA((4,)),
        semo=pltpu.SemaphoreType.DMA((4,)),
    )
    if first_layer:
        scratch["bdeg"] = pltpu.VMEM_SHARED((R,), jnp.float32)
        scratch["ddeg"] = pltpu.VMEM_SHARED((R,), jnp.float32)

    def body(*refs, in_table, edge_acc, ibufs, rowss, accbuf,
             degbuf, onesv, semi, semg, sems, semo, bdeg=None, ddeg=None):
        if first_layer:
            y_h, packed_h, h_out, binv_out, dinv_out = refs
        else:
            y_h, packed_h, binv_h, dinv_h, out_h = refs

        c = lax.axis_index("c")
        t = lax.axis_index("s")
        r0 = t * ROWS_PER_TILE
        cnt = base + jnp.where(t < extra, 1, 0)
        c0 = t * base + jnp.minimum(t, extra)
        col0 = c * Fc
        rslice = pl.ds(r0, ROWS_PER_TILE)

        def zrow_f(i, carry):
            for jj in range(Fc // 16):
                accbuf[i, pl.ds(jj * 16, 16)] = jnp.zeros((16,), jnp.float32)
            return carry

        lax.fori_loop(0, SROWS, zrow_f, 0)

        def zdeg_f(i, carry):
            degbuf[pl.ds(i * 16, 16)] = jnp.zeros((16,), jnp.float32)
            return carry

        lax.fori_loop(0, ROWS_PER_TILE // 16, zdeg_f, 0)

        if first_layer:
            for jj in range(chunk // 16):
                onesv[pl.ds(jj * 16, 16)] = jnp.ones((16,), jnp.float32)

        for sb in range(SUB):
            pltpu.sync_copy(accbuf, edge_acc.at[pl.ds(r0 + sb * SROWS, SROWS)])
        if first_layer:
            pltpu.sync_copy(degbuf, bdeg.at[rslice])
            pltpu.sync_copy(degbuf, ddeg.at[rslice])
        pltpu.sync_copy(y_h.at[rslice, pl.ds(col0, Fc)], in_table.at[rslice])
        plsc.subcore_barrier()

        def run_hop(gsrc, gi, sdst, si, deg_tbl, di):
            def issue_idx(j, k):
                pltpu.async_copy(
                    packed_h.at[pl.ds(2 * (c0 + j), 2)], ibufs.at[k],
                    semi.at[k],
                )

            def wait_idx(k):
                pltpu.make_async_copy(
                    packed_h.at[pl.ds(0, 2)], ibufs.at[k], semi.at[k]
                ).wait()

            def issue_scatter(k):
                pltpu.async_copy(
                    rowss.at[k], sdst.at[ibufs.at[k, si]], sems.at[k],
                    add=True,
                )
                if deg_tbl is not None:
                    pltpu.async_copy(
                        onesv, deg_tbl.at[ibufs.at[k, di]], semo.at[k],
                        add=True,
                    )

            def wait_scatter(k):
                pltpu.make_async_copy(
                    rowss.at[k], sdst.at[ibufs.at[k, si]], sems.at[k]
                ).wait()
                if deg_tbl is not None:
                    pltpu.make_async_copy(
                        onesv, deg_tbl.at[ibufs.at[k, di]], semo.at[k]
                    ).wait()

            issue_idx(0, 0)
            issue_idx(1, 1)

            def step4(g, carry):
                for k in range(4):
                    j = 4 * g + k
                    k1 = (k + 3) % 4
                    k2 = (k + 2) % 4

                    @pl.when(j < cnt)
                    def _(j=j, k=k):
                        wait_idx(k)
                        pltpu.async_copy(
                            gsrc.at[ibufs.at[k, gi]], rowss.at[k], semg.at[k]
                        )

                    @pl.when(jnp.logical_and(j >= 1, j <= cnt))
                    def _(j=j, k1=k1):
                        pltpu.make_async_copy(
                            gsrc.at[ibufs.at[k1, gi]], rowss.at[k1],
                            semg.at[k1],
                        ).wait()
                        issue_scatter(k1)

                    @pl.when(jnp.logical_and(j >= 2, j <= cnt + 1))
                    def _(j=j, k2=k2):
                        wait_scatter(k2)

                    @pl.when(j + 2 < cnt)
                    def _(j=j, k2=k2):
                        issue_idx(j + 2, k2)

                return carry

            lax.fori_loop(0, (base + 1 + 2 + 3) // 4, step4, 0)

        run_hop(in_table, 0, edge_acc, 1, bdeg if first_layer else None, 1)
        plsc.subcore_barrier()

        for sb in range(SUB):
            pltpu.sync_copy(accbuf, in_table.at[pl.ds(r0 + sb * SROWS, SROWS)])
        if first_layer:
            pltpu.sync_copy(bdeg.at[rslice], degbuf)

            def inv_g(g, carry):
                sl = pl.ds(g * 16, 16)
                d = degbuf[sl]
                degbuf[sl] = jnp.where(d > 0.0, 1.0 / d, 0.0)
                return carry

            lax.fori_loop(0, ROWS_PER_TILE // 16, inv_g, 0)
        else:
            pltpu.sync_copy(binv_h.at[rslice], degbuf)

        for sb in range(SUB):
            sbslice = pl.ds(r0 + sb * SROWS, SROWS)
            pltpu.sync_copy(edge_acc.at[sbslice], accbuf)

            def scale_r(g, carry, sb=sb):
                deg16 = degbuf[pl.ds(sb * SROWS + g * 16, 16)]
                for r16 in range(16):
                    bs = deg16[r16]
                    row = g * 16 + r16
                    for jj in range(Fc // 16):
                        sl = pl.ds(jj * 16, 16)
                        accbuf[row, sl] = accbuf[row, sl] * bs
                return carry

            lax.fori_loop(0, SROWS // 16, scale_r, 0)
            pltpu.sync_copy(accbuf, edge_acc.at[sbslice])
        if first_layer:

            @pl.when(c == 0)
            def _():
                pltpu.sync_copy(degbuf, binv_out.at[rslice])

        plsc.subcore_barrier()

        run_hop(edge_acc, 1, in_table, 0, ddeg if first_layer else None, 0)
        plsc.subcore_barrier()

        if first_layer:
            pltpu.sync_copy(ddeg.at[rslice], degbuf)

            def inv_g2(g, carry):
                sl = pl.ds(g * 16, 16)
                d = degbuf[sl]
                degbuf[sl] = jnp.where(d > 0.0, 1.0 / d, 0.0)
                return carry

            lax.fori_loop(0, ROWS_PER_TILE // 16, inv_g2, 0)
        else:
            pltpu.sync_copy(dinv_h.at[rslice], degbuf)

        dst = h_out if first_layer else out_h
        for sb in range(SUB):
            sbslice = pl.ds(r0 + sb * SROWS, SROWS)
            pltpu.sync_copy(in_table.at[sbslice], accbuf)

            def scale_out(g, carry, sb=sb):
                deg16 = degbuf[pl.ds(sb * SROWS + g * 16, 16)]
                for r16 in range(16):
                    ds_ = deg16[r16]
                    row = g * 16 + r16
                    for jj in range(Fc // 16):
                        sl = pl.ds(jj * 16, 16)
                        v = accbuf[row, sl] * ds_
                        if first_layer:
                            v = jnp.where(
                                v > 0.0, v, jnp.exp(jnp.minimum(v, 0.0)) - 1.0
                            )
                        accbuf[row, sl] = v
                return carry

            lax.fori_loop(0, SROWS // 16, scale_out, 0)
            pltpu.sync_copy(accbuf, dst.at[sbslice, pl.ds(col0, Fc)])
        if first_layer:

            @pl.when(c == 0)
            def _():
                pltpu.sync_copy(degbuf, dinv_out.at[rslice])

    return pl.kernel(
        body,
        out_type=out_type,
        mesh=mesh,
        scratch_types=scratch,
        compiler_params=_SC_PARAMS,
    )


@jax.jit
def kernel(x, hyperedge_index, W1, b1, W2, b2):
    node_idx = hyperedge_index[0]
    edge_idx = hyperedge_index[1]

    num_heads, in_dim, head_dim = W1.shape
    hidden = num_heads * head_dim
    num_classes = W2.shape[1]

    w1cat = W1.transpose(1, 0, 2).reshape(in_dim, hidden)
    b1cat = b1.reshape(1, hidden)
    w2p = jnp.pad(W2, ((0, 0), (0, 128 - num_classes)))
    b2p = jnp.pad(b2, (0, 128 - num_classes)).reshape(1, 128)

    def pack_idx(ch):
        nc = NNZ // ch
        return jnp.stack(
            [node_idx.reshape(nc, ch), edge_idx.reshape(nc, ch)], axis=1
        ).reshape(2 * nc, ch)

    packed1 = pack_idx(128)
    packed2 = pack_idx(256)

    xp = jnp.pad(x, ((0, R - x.shape[0]), (0, 0)))
    y1 = _matmul(xp, w1cat, b1cat, 640)
    agg1 = _make_agg(hidden // 2, True, 128)
    h_pad, binv, dinv = agg1(y1, packed1)

    y2 = _matmul(h_pad, w2p, b2p, 640)
    agg2 = _make_agg(num_classes // 2, False, 256)
    out_pad = agg2(y2, packed2, binv, dinv)

    return out_pad[:N_NODES, :num_classes]
